# baseline (speedup 1.0000x reference)
import jax
import jax.numpy as jnp
from jax.experimental import pallas as pl
from jax.experimental.pallas import tpu as pltpu

_B = 8
_C_OUT = 64
_STRIDE = 12
_N = _B * _C_OUT
_M = 16
_A = 8


def _ring_kernel(x_ref, o_ref, buf, in_sems, out_sems):
    ins = [
        pltpu.make_async_copy(
            x_ref.at[p // _C_OUT, (p % _C_OUT) * _STRIDE],
            buf.at[p % _M],
            in_sems.at[p % _M],
        )
        for p in range(_N)
    ]
    outs = [
        pltpu.make_async_copy(
            buf.at[p % _M],
            o_ref.at[p // _C_OUT, p % _C_OUT],
            out_sems.at[p % _M],
        )
        for p in range(_N)
    ]
    for i in range(_N + _A):
        if i < _N:
            if i >= _M:
                outs[i - _M].wait()
            ins[i].start()
        j = i - _A
        if 0 <= j < _N:
            ins[j].wait()
            outs[j].start()
    for p in range(_N - _M, _N):
        outs[p].wait()


def kernel(x):
    return pl.pallas_call(
        _ring_kernel,
        in_specs=[pl.BlockSpec(memory_space=pl.ANY)],
        out_specs=pl.BlockSpec(memory_space=pl.ANY),
        out_shape=jax.ShapeDtypeStruct((_B, _C_OUT, 224, 224), x.dtype),
        scratch_shapes=[
            pltpu.VMEM((_M, 224, 224), jnp.float32),
            pltpu.SemaphoreType.DMA((_M,)),
            pltpu.SemaphoreType.DMA((_M,)),
        ],
    )(x)
